# baseline (speedup 1.0000x reference)
import jax
import jax.numpy as jnp
from jax import lax
from jax.experimental import pallas as pl
from jax.experimental.pallas import tpu as pltpu

_KC = 128


def _vq_batch(ztb, cbt, cn, iota):
    bz = ztb.shape[1]
    k = cbt.shape[1]
    zn = jnp.sum(ztb * ztb, axis=0, keepdims=True)
    ztb2 = -2.0 * ztb
    m_run = jnp.full((1, bz), jnp.inf, jnp.float32)
    idx_run = jnp.zeros((1, bz), jnp.int32)
    for c in range(k // _KC):
        cbt_c = cbt[:, c * _KC:(c + 1) * _KC]
        cn_c = cn[c * _KC:(c + 1) * _KC]
        dot_c = lax.dot_general(cbt_c, ztb2, (((0,), (0,)), ((), ())),
                                preferred_element_type=jnp.float32)
        d2 = zn + dot_c + cn_c
        m_c = jnp.min(d2, axis=0, keepdims=True)
        idx_c = (jnp.argmin(d2, axis=0).astype(jnp.int32)[None, :]
                 + jnp.int32(c * _KC))
        better = m_c < m_run
        idx_run = jnp.where(better, idx_c, idx_run)
        m_run = jnp.minimum(m_run, m_c)
    acc = jnp.zeros((cbt.shape[0], bz), jnp.float32)
    for c in range(k // _KC):
        cbt_c = cbt[:, c * _KC:(c + 1) * _KC]
        onehot = (iota == idx_run - jnp.int32(c * _KC)).astype(jnp.float32)
        acc = acc + lax.dot_general(cbt_c, onehot, (((1,), (0,)), ((), ())),
                                    preferred_element_type=jnp.float32)
    return acc


def _vq_body(zt_hbm, cbt_ref, out_hbm, zbuf, obuf, insem, outsem):
    cbt = cbt_ref[...]
    nb = zt_hbm.shape[0]
    bz = zt_hbm.shape[2]
    cn = jnp.sum(cbt * cbt, axis=0, keepdims=True).T
    iota = lax.broadcasted_iota(jnp.int32, (_KC, bz), 0)

    pltpu.make_async_copy(zt_hbm.at[0], zbuf.at[0], insem.at[0]).start()
    for b in range(nb):
        if b + 1 < nb:
            s_next = (b + 1) % 2
            pltpu.make_async_copy(zt_hbm.at[b + 1], zbuf.at[s_next],
                                  insem.at[s_next]).start()
        s = b % 2
        pltpu.make_async_copy(zt_hbm.at[b], zbuf.at[s], insem.at[s]).wait()
        acc = _vq_batch(zbuf[s], cbt, cn, iota)
        if b >= 2:
            pltpu.make_async_copy(obuf.at[s], out_hbm.at[b - 2],
                                  outsem.at[s]).wait()
        obuf[s] = acc
        pltpu.make_async_copy(obuf.at[s], out_hbm.at[b], outsem.at[s]).start()
    for b in range(max(nb - 2, 0), nb):
        s = b % 2
        pltpu.make_async_copy(obuf.at[s], out_hbm.at[b], outsem.at[s]).wait()


def _vq_tc(zt, cbt, interpret=False):
    nb, d, b = zt.shape
    return pl.pallas_call(
        _vq_body,
        in_specs=[
            pl.BlockSpec(memory_space=pl.ANY),
            pl.BlockSpec(memory_space=pltpu.MemorySpace.VMEM),
        ],
        out_specs=pl.BlockSpec(memory_space=pl.ANY),
        out_shape=jax.ShapeDtypeStruct((nb, d, b), jnp.float32),
        scratch_shapes=[
            pltpu.VMEM((2, d, b), jnp.float32),
            pltpu.VMEM((2, d, b), jnp.float32),
            pltpu.SemaphoreType.DMA((2,)),
            pltpu.SemaphoreType.DMA((2,)),
        ],
        interpret=interpret,
    )(zt, cbt)


def kernel(z, codebook):
    zt = jnp.swapaxes(z, 1, 2)
    out_t = _vq_tc(zt, codebook.T)
    return jnp.swapaxes(out_t, 1, 2)


# speedup vs baseline: 1.4512x; 1.4512x over previous
import jax
import jax.numpy as jnp
from jax import lax
from jax.experimental import pallas as pl

_KC = 128


def _vq_body(zt_ref, cbt_ref, out_ref):
    cbt = cbt_ref[...]
    k = cbt.shape[1]
    nb = zt_ref.shape[0]
    bz = zt_ref.shape[2]
    cn = jnp.sum(cbt * cbt, axis=0, keepdims=True).T
    iota = lax.broadcasted_iota(jnp.int32, (_KC, bz), 0)

    for b in range(nb):
        ztb = zt_ref[b]
        zn = jnp.sum(ztb * ztb, axis=0, keepdims=True)
        ztb2 = -2.0 * ztb
        m_run = None
        idx_run = None
        for c in range(k // _KC):
            cbt_c = cbt[:, c * _KC:(c + 1) * _KC]
            cn_c = cn[c * _KC:(c + 1) * _KC]
            dot_c = lax.dot_general(cbt_c, ztb2, (((0,), (0,)), ((), ())),
                                    preferred_element_type=jnp.float32)
            d2 = zn + dot_c + cn_c
            m_c = jnp.min(d2, axis=0, keepdims=True)
            idx_c = (jnp.argmin(d2, axis=0).astype(jnp.int32)[None, :]
                     + jnp.int32(c * _KC))
            if c == 0:
                m_run, idx_run = m_c, idx_c
            else:
                better = m_c < m_run
                idx_run = jnp.where(better, idx_c, idx_run)
                m_run = jnp.minimum(m_run, m_c)
        acc = None
        for c in range(k // _KC):
            cbt_c = cbt[:, c * _KC:(c + 1) * _KC]
            onehot = (iota == idx_run - jnp.int32(c * _KC)).astype(jnp.float32)
            part = lax.dot_general(cbt_c, onehot, (((1,), (0,)), ((), ())),
                                   preferred_element_type=jnp.float32)
            acc = part if c == 0 else acc + part
        out_ref[b] = acc


def _vq_tc(zt, cbt, interpret=False):
    nb, d, b = zt.shape
    return pl.pallas_call(
        _vq_body,
        out_shape=jax.ShapeDtypeStruct((nb, d, b), jnp.float32),
        interpret=interpret,
    )(zt, cbt)


def kernel(z, codebook):
    zt = jnp.swapaxes(z, 1, 2)
    out_t = _vq_tc(zt, codebook.T)
    return jnp.swapaxes(out_t, 1, 2)
